# baseline (speedup 1.0000x reference)
import functools

import jax
import jax.numpy as jnp
from jax import lax
from jax.experimental import pallas as pl
from jax.experimental.pallas import tpu as pltpu
from jax.experimental.pallas import tpu_sc as plsc

VOCAB_ROWS = 1000000
B_DIM, T_DIM = 4096, 200
B = B_DIM * T_DIM
D = 64
NC, NS = 2, 16
NW = NC * NS
B_W = B // NW
CHUNK = 128
C_W = B_W // CHUNK
G = 5
ROWS_G = CHUNK * G
N_GRP = C_W // G
N_PAIR = N_GRP // 2


NBLK = VOCAB_ROWS // CHUNK
N_MAIN_BLK = NBLK - 1
N_EXTRA = N_MAIN_BLK - 244 * NW
TAIL_COL = VOCAB_ROWS - CHUNK
TAIL_WORKER = 4


def _sc_convert(table_t, tail_t):
    mesh = plsc.VectorSubcoreMesh(core_axis_name="c", subcore_axis_name="s")

    @functools.partial(
        pl.kernel,
        out_type=jax.ShapeDtypeStruct((VOCAB_ROWS * D,), jnp.float32),
        mesh=mesh,
        compiler_params=pltpu.CompilerParams(needs_layout_passes=False),
        scratch_types=[
            pltpu.VMEM((2, D, CHUNK), jnp.float32),
            pltpu.VMEM((2, D * CHUNK), jnp.float32),
            pltpu.SemaphoreType.DMA,
            pltpu.SemaphoreType.DMA,
            pltpu.SemaphoreType.DMA,
            pltpu.SemaphoreType.DMA,
        ],
    )
    def k(tt_hbm, tail_hbm, out_hbm, in_v, out_v, gi0, gi1, go0, go1):
        wid = lax.axis_index("s") * NC + lax.axis_index("c")
        gsem = (gi0, gi1)
        osem = (go0, go1)
        iota = lax.iota(jnp.int32, 16)

        extra = wid < N_EXTRA
        n_my = 244 + jnp.where(extra, 1, 0)

        def blk(t):
            return wid + t * NW

        def in_copy(t, b):
            return pltpu.async_copy(
                tt_hbm.at[:, pl.ds(blk(t) * CHUNK, CHUNK)],
                in_v.at[b],
                gsem[b],
            )

        def wait_in(b):
            pltpu.make_async_copy(
                tt_hbm.at[:, pl.ds(0, CHUNK)], in_v.at[b], gsem[b]
            ).wait()

        BLK_ELEMS = D * CHUNK

        def out_copy(t, b):
            return pltpu.async_copy(
                out_v.at[b],
                out_hbm.at[pl.ds(blk(t) * BLK_ELEMS, BLK_ELEMS)],
                osem[b],
            )

        def wait_out(b):
            pltpu.make_async_copy(
                out_v.at[b], out_hbm.at[pl.ds(0, BLK_ELEMS)], osem[b]
            ).wait()

        rowv = [dk * 16 + iota for dk in range(4)]
        zero16 = iota * 0

        def transpose(b):
            @plsc.parallel_loop(0, CHUNK // 16, unroll=2)
            def _(c0):
                cbase = zero16 + c0 * 16
                obase = c0 * 16 * D
                for cc in range(16):
                    cvec = cbase + cc
                    for dk in range(4):
                        vec = plsc.load_gather(in_v.at[b], [rowv[dk], cvec])
                        out_v[b, pl.ds(obase + (cc * D + dk * 16), 16)] = vec

        in_copy(0, 0)

        @pl.loop(0, 122)
        def _(tp):
            for b in range(2):
                t = 2 * tp + b

                @pl.when(t + 1 < n_my)
                def _():
                    in_copy(t + 1, 1 - b)

                @pl.when(t > 1)
                def _():
                    wait_out(b)

                wait_in(b)
                transpose(b)
                out_copy(t, b)

        @pl.when(extra)
        def _():
            wait_out(0)
            wait_in(0)
            transpose(0)
            out_copy(244, 0)

        wait_out(0)
        wait_out(1)

        @pl.when(wid == TAIL_WORKER)
        def _():
            pltpu.sync_copy(
                tt_hbm.at[:, pl.ds(N_MAIN_BLK * CHUNK, CHUNK)], in_v.at[0]
            )
            transpose(0)
            pltpu.sync_copy(
                out_v.at[0], out_hbm.at[pl.ds(N_MAIN_BLK * BLK_ELEMS, BLK_ELEMS)]
            )
            pltpu.sync_copy(tail_hbm, in_v.at[0])
            transpose(0)
            pltpu.sync_copy(
                out_v.at[0], out_hbm.at[pl.ds(TAIL_COL * D, BLK_ELEMS)]
            )

    return k(table_t, tail_t)


def _sc_gather(idx2d, table2m):
    mesh = plsc.VectorSubcoreMesh(core_axis_name="c", subcore_axis_name="s")

    @functools.partial(
        pl.kernel,
        out_type=jax.ShapeDtypeStruct((B, 2 * D), jnp.float32),
        mesh=mesh,
        compiler_params=pltpu.CompilerParams(use_tc_tiling_on_sc=False),
        scratch_types=[
            pltpu.VMEM((C_W, CHUNK), jnp.int32),
            pltpu.VMEM((2, ROWS_G, D), jnp.float32),
            pltpu.SemaphoreType.DMA,
            pltpu.SemaphoreType.DMA,
            pltpu.SemaphoreType.DMA,
            pltpu.SemaphoreType.DMA,
        ],
    )
    def k(idx_hbm, table_hbm, out_hbm, idx_v, rows_v, gs0, gs1, os0, os1):
        wid = lax.axis_index("s") * NC + lax.axis_index("c")
        row_base = wid * B_W
        chunk_base = wid * C_W
        gsem = (gs0, gs1)
        osem = (os0, os1)

        pltpu.sync_copy(idx_hbm.at[pl.ds(chunk_base, C_W)], idx_v)

        def out_slice(t, b):
            return out_hbm.at[
                pl.ds(row_base + (2 * t + b) * ROWS_G, ROWS_G), pl.ds(0, D)
            ]

        def fire_gathers(t, b):
            return [
                pltpu.async_copy(
                    table_hbm.at[idx_v.at[(2 * t + b) * G + j]],
                    rows_v.at[b, pl.ds(j * CHUNK, CHUNK)],
                    gsem[b],
                )
                for j in range(G)
            ]

        @pl.loop(0, N_PAIR)
        def _(t):
            @pl.when(t > 0)
            def _():
                pltpu.make_async_copy(rows_v.at[0], out_slice(t, 0), osem[0]).wait()

            d0 = fire_gathers(t, 0)

            @pl.when(t > 0)
            def _():
                pltpu.make_async_copy(rows_v.at[1], out_slice(t, 1), osem[1]).wait()

            d1 = fire_gathers(t, 1)
            for c in d0:
                c.wait()
            pltpu.async_copy(rows_v.at[0], out_slice(t, 0), osem[0])
            for c in d1:
                c.wait()
            pltpu.async_copy(rows_v.at[1], out_slice(t, 1), osem[1])

        for b in range(2):
            pltpu.make_async_copy(
                rows_v.at[b], out_slice(N_PAIR - 1, b), osem[b]
            ).wait()

    return k(idx2d, table2m)


def kernel(x, table):
    table_t = table.T
    tail_t = lax.slice(table_t, (0, TAIL_COL), (D, VOCAB_ROWS))
    tflat = _sc_convert(table_t, tail_t)
    table_rm = tflat.reshape(VOCAB_ROWS, D)
    idx2 = x[:, :, 1].astype(jnp.int32).reshape(B // CHUNK, CHUNK)
    out = _sc_gather(idx2, table_rm)
    return out[:, :D].reshape(B_DIM, T_DIM, D)


# speedup vs baseline: 1.2131x; 1.2131x over previous
import functools

import jax
import jax.numpy as jnp
from jax import lax
from jax.experimental import pallas as pl
from jax.experimental.pallas import tpu as pltpu
from jax.experimental.pallas import tpu_sc as plsc

VOCAB_ROWS = 1000000
B_DIM, T_DIM = 4096, 200
B = B_DIM * T_DIM
D = 64
NC, NS = 2, 16
NW = NC * NS
B_W = B // NW
CHUNK = 128
C_W = B_W // CHUNK
G = 5
ROWS_G = CHUNK * G
N_GRP = C_W // G
N_PAIR = N_GRP // 2


NBLK = VOCAB_ROWS // CHUNK
N_MAIN_BLK = NBLK - 1
N_EXTRA = N_MAIN_BLK - 244 * NW
TAIL_COL = VOCAB_ROWS - CHUNK
TAIL_WORKER = 4


def _sc_convert(table_t, tail_t):
    mesh = plsc.VectorSubcoreMesh(core_axis_name="c", subcore_axis_name="s")

    @functools.partial(
        pl.kernel,
        out_type=jax.ShapeDtypeStruct((VOCAB_ROWS * D,), jnp.float32),
        mesh=mesh,
        compiler_params=pltpu.CompilerParams(needs_layout_passes=False),
        scratch_types=[
            pltpu.VMEM((2, D, CHUNK + 1), jnp.float32),
            pltpu.VMEM((2, D * CHUNK), jnp.float32),
            pltpu.SemaphoreType.DMA,
            pltpu.SemaphoreType.DMA,
            pltpu.SemaphoreType.DMA,
            pltpu.SemaphoreType.DMA,
        ],
    )
    def k(tt_hbm, tail_hbm, out_hbm, in_v, out_v, gi0, gi1, go0, go1):
        wid = lax.axis_index("s") * NC + lax.axis_index("c")
        gsem = (gi0, gi1)
        osem = (go0, go1)
        iota = lax.iota(jnp.int32, 16)

        extra = wid < N_EXTRA
        n_my = 244 + jnp.where(extra, 1, 0)

        def blk(t):
            return wid + t * NW

        def in_copy(t, b):
            return pltpu.async_copy(
                tt_hbm.at[:, pl.ds(blk(t) * CHUNK, CHUNK)],
                in_v.at[b, :, pl.ds(0, CHUNK)],
                gsem[b],
            )

        def wait_in(b):
            pltpu.make_async_copy(
                tt_hbm.at[:, pl.ds(0, CHUNK)],
                in_v.at[b, :, pl.ds(0, CHUNK)],
                gsem[b],
            ).wait()

        BLK_ELEMS = D * CHUNK

        def out_copy(t, b):
            return pltpu.async_copy(
                out_v.at[b],
                out_hbm.at[pl.ds(blk(t) * BLK_ELEMS, BLK_ELEMS)],
                osem[b],
            )

        def wait_out(b):
            pltpu.make_async_copy(
                out_v.at[b], out_hbm.at[pl.ds(0, BLK_ELEMS)], osem[b]
            ).wait()

        rowv = [dk * 16 + iota for dk in range(4)]
        zero16 = iota * 0

        def transpose(b):
            @plsc.parallel_loop(0, CHUNK, unroll=8)
            def _(c):
                cvec = zero16 + c
                base = c * D
                for dk in range(4):
                    vec = plsc.load_gather(in_v.at[b], [rowv[dk], cvec])
                    out_v[b, pl.ds(base + dk * 16, 16)] = vec

        in_copy(0, 0)

        @pl.loop(0, 122)
        def _(tp):
            for b in range(2):
                t = 2 * tp + b

                @pl.when(t + 1 < n_my)
                def _():
                    in_copy(t + 1, 1 - b)

                @pl.when(t > 1)
                def _():
                    wait_out(b)

                wait_in(b)
                transpose(b)
                out_copy(t, b)

        @pl.when(extra)
        def _():
            wait_out(0)
            wait_in(0)
            transpose(0)
            out_copy(244, 0)

        wait_out(0)
        wait_out(1)

        @pl.when(wid == TAIL_WORKER)
        def _():
            pltpu.sync_copy(
                tt_hbm.at[:, pl.ds(N_MAIN_BLK * CHUNK, CHUNK)],
                in_v.at[0, :, pl.ds(0, CHUNK)],
            )
            transpose(0)
            pltpu.sync_copy(
                out_v.at[0], out_hbm.at[pl.ds(N_MAIN_BLK * BLK_ELEMS, BLK_ELEMS)]
            )
            pltpu.sync_copy(tail_hbm, in_v.at[0, :, pl.ds(0, CHUNK)])
            transpose(0)
            pltpu.sync_copy(
                out_v.at[0], out_hbm.at[pl.ds(TAIL_COL * D, BLK_ELEMS)]
            )

    return k(table_t, tail_t)


def _sc_gather(idx2d, table2m):
    mesh = plsc.VectorSubcoreMesh(core_axis_name="c", subcore_axis_name="s")

    @functools.partial(
        pl.kernel,
        out_type=jax.ShapeDtypeStruct((B, 2 * D), jnp.float32),
        mesh=mesh,
        compiler_params=pltpu.CompilerParams(use_tc_tiling_on_sc=False),
        scratch_types=[
            pltpu.VMEM((C_W, CHUNK), jnp.int32),
            pltpu.VMEM((2, ROWS_G, D), jnp.float32),
            pltpu.SemaphoreType.DMA,
            pltpu.SemaphoreType.DMA,
            pltpu.SemaphoreType.DMA,
            pltpu.SemaphoreType.DMA,
        ],
    )
    def k(idx_hbm, table_hbm, out_hbm, idx_v, rows_v, gs0, gs1, os0, os1):
        wid = lax.axis_index("s") * NC + lax.axis_index("c")
        row_base = wid * B_W
        chunk_base = wid * C_W
        gsem = (gs0, gs1)
        osem = (os0, os1)

        pltpu.sync_copy(idx_hbm.at[pl.ds(chunk_base, C_W)], idx_v)

        def out_slice(t, b):
            return out_hbm.at[
                pl.ds(row_base + (2 * t + b) * ROWS_G, ROWS_G), pl.ds(0, D)
            ]

        def fire_gathers(t, b):
            return [
                pltpu.async_copy(
                    table_hbm.at[idx_v.at[(2 * t + b) * G + j]],
                    rows_v.at[b, pl.ds(j * CHUNK, CHUNK)],
                    gsem[b],
                )
                for j in range(G)
            ]

        @pl.loop(0, N_PAIR)
        def _(t):
            @pl.when(t > 0)
            def _():
                pltpu.make_async_copy(rows_v.at[0], out_slice(t, 0), osem[0]).wait()

            d0 = fire_gathers(t, 0)

            @pl.when(t > 0)
            def _():
                pltpu.make_async_copy(rows_v.at[1], out_slice(t, 1), osem[1]).wait()

            d1 = fire_gathers(t, 1)
            for c in d0:
                c.wait()
            pltpu.async_copy(rows_v.at[0], out_slice(t, 0), osem[0])
            for c in d1:
                c.wait()
            pltpu.async_copy(rows_v.at[1], out_slice(t, 1), osem[1])

        for b in range(2):
            pltpu.make_async_copy(
                rows_v.at[b], out_slice(N_PAIR - 1, b), osem[b]
            ).wait()

    return k(idx2d, table2m)


def kernel(x, table):
    table_t = table.T
    tail_t = lax.slice(table_t, (0, TAIL_COL), (D, VOCAB_ROWS))
    tflat = _sc_convert(table_t, tail_t)
    table_rm = tflat.reshape(VOCAB_ROWS, D)
    idx2 = x[:, :, 1].astype(jnp.int32).reshape(B // CHUNK, CHUNK)
    out = _sc_gather(idx2, table_rm)
    return out[:, :D].reshape(B_DIM, T_DIM, D)


# speedup vs baseline: 1.6928x; 1.3954x over previous
import functools

import jax
import jax.numpy as jnp
from jax import lax
from jax.experimental import pallas as pl
from jax.experimental.pallas import tpu as pltpu
from jax.experimental.pallas import tpu_sc as plsc

VOCAB_ROWS = 1000000
B_DIM, T_DIM = 4096, 200
B = B_DIM * T_DIM
D = 64
NC, NS = 2, 16
NW = NC * NS
B_W = B // NW
CHUNK = 128
C_W = B_W // CHUNK
G = 5
ROWS_G = CHUNK * G
N_GRP = C_W // G
N_PAIR = N_GRP // 2


def _sc_gather(idx2d, table2m):
    mesh = plsc.VectorSubcoreMesh(core_axis_name="c", subcore_axis_name="s")

    @functools.partial(
        pl.kernel,
        out_type=jax.ShapeDtypeStruct((B, 2 * D), jnp.float32),
        mesh=mesh,
        compiler_params=pltpu.CompilerParams(use_tc_tiling_on_sc=False),
        scratch_types=[
            pltpu.VMEM((C_W, CHUNK), jnp.int32),
            pltpu.VMEM((2, ROWS_G, D), jnp.float32),
            pltpu.SemaphoreType.DMA,
            pltpu.SemaphoreType.DMA,
            pltpu.SemaphoreType.DMA,
            pltpu.SemaphoreType.DMA,
        ],
    )
    def k(idx_hbm, table_hbm, out_hbm, idx_v, rows_v, gs0, gs1, os0, os1):
        wid = lax.axis_index("s") * NC + lax.axis_index("c")
        row_base = wid * B_W
        chunk_base = wid * C_W
        gsem = (gs0, gs1)
        osem = (os0, os1)

        pltpu.sync_copy(idx_hbm.at[pl.ds(chunk_base, C_W)], idx_v)

        def out_slice(t, b):
            return out_hbm.at[
                pl.ds(row_base + (2 * t + b) * ROWS_G, ROWS_G), pl.ds(0, D)
            ]

        def fire_gathers(t, b):
            return [
                pltpu.async_copy(
                    table_hbm.at[idx_v.at[(2 * t + b) * G + j]],
                    rows_v.at[b, pl.ds(j * CHUNK, CHUNK)],
                    gsem[b],
                )
                for j in range(G)
            ]

        @pl.loop(0, N_PAIR)
        def _(t):
            @pl.when(t > 0)
            def _():
                pltpu.make_async_copy(rows_v.at[0], out_slice(t, 0), osem[0]).wait()

            d0 = fire_gathers(t, 0)

            @pl.when(t > 0)
            def _():
                pltpu.make_async_copy(rows_v.at[1], out_slice(t, 1), osem[1]).wait()

            d1 = fire_gathers(t, 1)
            for c in d0:
                c.wait()
            pltpu.async_copy(rows_v.at[0], out_slice(t, 0), osem[0])
            for c in d1:
                c.wait()
            pltpu.async_copy(rows_v.at[1], out_slice(t, 1), osem[1])

        for b in range(2):
            pltpu.make_async_copy(
                rows_v.at[b], out_slice(N_PAIR - 1, b), osem[b]
            ).wait()

    return k(idx2d, table2m)


def kernel(x, table):
    tpad = jnp.pad(table, ((0, 0), (0, D)))
    table2m = tpad.reshape(2 * VOCAB_ROWS, D)
    idx2 = (x[:, :, 1].astype(jnp.int32) * 2).reshape(B // CHUNK, CHUNK)
    out = _sc_gather(idx2, table2m)
    return out[:, :D].reshape(B_DIM, T_DIM, D)
